# Initial kernel scaffold; baseline (speedup 1.0000x reference)
import jax
import jax.numpy as jnp
from jax.experimental import pallas as pl


def kernel(x, edge_attr, edges, W_m1, b_m1, W_m2, b_m2, W_u1, b_u1, W_u2, b_u2):
    raise NotImplementedError("write your pallas kernel here")


# speedup vs baseline: 2.8445x; 2.8445x over previous
import functools

import jax
import jax.numpy as jnp
from jax import lax
from jax.experimental import pallas as pl
from jax.experimental.pallas import tpu as pltpu
from jax.experimental.pallas import tpu_sc as plsc

N = 10000
E = 320000
D = 128
L = 16
NC = 2
NS = 16
NW = NC * NS
EPW = E // NW
CH = 80
NCH = EPW // CH
CW = 16
NPAD = 10240
NPT = NPAD // NS

_MESH = dict(core_axis_name="c", subcore_axis_name="s")


@functools.partial(
    pl.kernel,
    mesh=plsc.VectorSubcoreMesh(**_MESH),
    out_type=(jax.ShapeDtypeStruct((E, D), jnp.float32),
              jax.ShapeDtypeStruct((NC, NPAD, D), jnp.float32)),
    scratch_types=[
        pltpu.VMEM((CH,), jnp.int32),
        pltpu.VMEM((CH,), jnp.int32),
        pltpu.VMEM((CH, D), jnp.float32),
        pltpu.VMEM((CH, D), jnp.float32),
        pltpu.VMEM((CH, D), jnp.float32),
        pltpu.VMEM_SHARED((NPAD, D), jnp.float32),
        pltpu.SemaphoreType.DMA,
        pltpu.SemaphoreType.DMA,
    ],
)
def _sc_gather(xs_hbm, xr_hbm, send_hbm, recv_hbm, g_hbm, cnt_hbm,
               sidx, ridx, bufa, bufb, ones, scnt, sema, semb):
    cid = lax.axis_index("c")
    sid = lax.axis_index("s")
    wid = sid * NC + cid
    base = wid * EPW
    zero16 = jnp.zeros((L,), jnp.float32)
    one16 = jnp.ones((L,), jnp.float32)

    def fill_body(r, c):
        for j in range(D // L):
            bufa[r, pl.ds(j * L, L)] = zero16
            ones[r, pl.ds(j * L, L)] = one16
        return c

    lax.fori_loop(0, CH, fill_body, 0)

    nbase = sid * NPT
    for k in range(NPT // CH):
        pltpu.sync_copy(bufa, scnt.at[pl.ds(nbase + k * CH, CH)])
    plsc.subcore_barrier()

    def chunk_body(ci, carry):
        off = pl.multiple_of(base + ci * CH, 8)
        pltpu.sync_copy(send_hbm.at[pl.ds(off, CH)], sidx)
        pltpu.sync_copy(recv_hbm.at[pl.ds(off, CH)], ridx)
        cpa = pltpu.async_copy(xs_hbm.at[sidx], bufa, sema)
        cpb = pltpu.async_copy(xr_hbm.at[ridx], bufb, semb)
        cpa.wait()
        cpb.wait()
        pltpu.sync_copy(ones, scnt.at[ridx], add=True)

        def row_body(r, c2):
            for j in range(D // L):
                sl = pl.ds(j * L, L)
                bufa[r, sl] = bufa[r, sl] + bufb[r, sl]
            return c2

        lax.fori_loop(0, CH, row_body, 0)
        pltpu.sync_copy(bufa, g_hbm.at[pl.ds(off, CH)])
        return carry

    lax.fori_loop(0, NCH, chunk_body, 0)
    plsc.subcore_barrier()

    for k in range(NPT // CH):
        sl = pl.ds(nbase + k * CH, CH)
        pltpu.sync_copy(scnt.at[sl], bufa)
        pltpu.sync_copy(bufa, cnt_hbm.at[cid, sl])


@functools.partial(
    pl.kernel,
    mesh=plsc.VectorSubcoreMesh(**_MESH),
    out_type=jax.ShapeDtypeStruct((NC, NPAD, D), jnp.float32),
    scratch_types=[
        pltpu.VMEM((CH,), jnp.int32),
        pltpu.VMEM((CH, D), jnp.float32),
        pltpu.VMEM_SHARED((NPAD, D), jnp.float32),
        pltpu.SemaphoreType.DMA,
    ],
)
def _sc_scatter(h_hbm, recv_hbm, sum_hbm, ridx, hbuf, ssum, sem):
    cid = lax.axis_index("c")
    sid = lax.axis_index("s")
    zero16 = jnp.zeros((L,), jnp.float32)

    def zrow_body(r, c):
        for j in range(D // L):
            hbuf[r, pl.ds(j * L, L)] = zero16
        return c

    lax.fori_loop(0, CH, zrow_body, 0)

    nbase = sid * NPT
    for k in range(NPT // CH):
        pltpu.sync_copy(hbuf, ssum.at[pl.ds(nbase + k * CH, CH)])
    plsc.subcore_barrier()

    base = (sid * NC + cid) * EPW

    def chunk_body(ci, c):
        off = pl.multiple_of(base + ci * CH, 8)
        pltpu.sync_copy(recv_hbm.at[pl.ds(off, CH)], ridx)
        pltpu.async_copy(h_hbm.at[pl.ds(off, CH)], hbuf, sem).wait()
        pltpu.sync_copy(hbuf, ssum.at[ridx], add=True)
        return c

    lax.fori_loop(0, NCH, chunk_body, 0)
    plsc.subcore_barrier()

    for k in range(NPT // CH):
        sl = pl.ds(nbase + k * CH, CH)
        pltpu.sync_copy(ssum.at[sl], hbuf)
        pltpu.sync_copy(hbuf, sum_hbm.at[cid, sl])


def _tc_node_transform(x2, wa, wb):
    TN = 2000

    def body(x_ref, wa_ref, wb_ref, xs_ref, xr_ref):
        xv = x_ref[...]
        xs_ref[...] = jnp.dot(xv, wa_ref[...], preferred_element_type=jnp.float32)
        xr_ref[...] = jnp.dot(xv, wb_ref[...], preferred_element_type=jnp.float32)

    return pl.pallas_call(
        body,
        grid=(N // TN,),
        in_specs=[
            pl.BlockSpec((TN, D), lambda i: (i, 0)),
            pl.BlockSpec((D, D), lambda i: (0, 0)),
            pl.BlockSpec((D, D), lambda i: (0, 0)),
        ],
        out_specs=[
            pl.BlockSpec((TN, D), lambda i: (i, 0)),
            pl.BlockSpec((TN, D), lambda i: (i, 0)),
        ],
        out_shape=[jax.ShapeDtypeStruct((N, D), jnp.float32)] * 2,
    )(x2, wa, wb)


def _tc_message(g, ea2, wc, b1, w2, b2):
    TE = 2000

    def body(g_ref, ea_ref, wc_ref, b1_ref, w2_ref, b2_ref, h_ref):
        pre = (g_ref[...]
               + jnp.dot(ea_ref[...], wc_ref[...], preferred_element_type=jnp.float32)
               + b1_ref[...])
        h1 = pre * jax.nn.sigmoid(pre)
        pre2 = jnp.dot(h1, w2_ref[...], preferred_element_type=jnp.float32) + b2_ref[...]
        h_ref[...] = pre2 * jax.nn.sigmoid(pre2)

    return pl.pallas_call(
        body,
        grid=(E // TE,),
        in_specs=[
            pl.BlockSpec((TE, D), lambda i: (i, 0)),
            pl.BlockSpec((TE, D), lambda i: (i, 0)),
            pl.BlockSpec((D, D), lambda i: (0, 0)),
            pl.BlockSpec((1, D), lambda i: (0, 0)),
            pl.BlockSpec((D, D), lambda i: (0, 0)),
            pl.BlockSpec((1, D), lambda i: (0, 0)),
        ],
        out_specs=pl.BlockSpec((TE, D), lambda i: (i, 0)),
        out_shape=jax.ShapeDtypeStruct((E, D), jnp.float32),
    )(g, ea2, wc, b1, w2, b2)


def _tc_update(x2, psum, pcnt, wu1, bu1, wu2, bu2):
    TN = 2000

    def body(x_ref, ps_ref, pc_ref, wu1_ref, bu1_ref, wu2_ref, bu2_ref, o_ref):
        s = ps_ref[0] + ps_ref[1]
        c = pc_ref[0, :, 0:1] + pc_ref[1, :, 0:1]
        agg = s / jnp.maximum(c, 1.0)
        xn = x_ref[...] + agg
        t = jnp.dot(xn, wu1_ref[...], preferred_element_type=jnp.float32) + bu1_ref[...]
        t = t * jax.nn.sigmoid(t)
        u = jnp.dot(t, wu2_ref[...], preferred_element_type=jnp.float32) + bu2_ref[...]
        o_ref[...] = xn + u

    return pl.pallas_call(
        body,
        grid=(N // TN,),
        in_specs=[
            pl.BlockSpec((TN, D), lambda i: (i, 0)),
            pl.BlockSpec((NC, TN, D), lambda i: (0, i, 0)),
            pl.BlockSpec((NC, TN, D), lambda i: (0, i, 0)),
            pl.BlockSpec((D, 2 * D), lambda i: (0, 0)),
            pl.BlockSpec((1, 2 * D), lambda i: (0, 0)),
            pl.BlockSpec((2 * D, D), lambda i: (0, 0)),
            pl.BlockSpec((1, D), lambda i: (0, 0)),
        ],
        out_specs=pl.BlockSpec((TN, D), lambda i: (i, 0)),
        out_shape=jax.ShapeDtypeStruct((N, D), jnp.float32),
    )(x2, psum, pcnt, wu1, bu1, wu2, bu2)


def kernel(x, edge_attr, edges, W_m1, b_m1, W_m2, b_m2, W_u1, b_u1, W_u2, b_u2):
    x2 = x[0]
    ea2 = edge_attr[0]
    send = edges[0]
    recv = edges[1]
    wa = W_m1[:D]
    wb = W_m1[D:2 * D]
    wc = W_m1[2 * D:]
    b1 = b_m1.reshape(1, D)
    b2 = b_m2.reshape(1, D)
    bu1 = b_u1.reshape(1, 2 * D)
    bu2 = b_u2.reshape(1, D)

    xs, xr = _tc_node_transform(x2, wa, wb)
    g, pcnt = _sc_gather(xs, xr, send, recv)
    h = _tc_message(g, ea2, wc, b1, W_m2, b2)
    psum = _sc_scatter(h, recv)
    xo = _tc_update(x2, psum, pcnt, W_u1, bu1, W_u2, bu2)
    return (xo[None], h[None])


# speedup vs baseline: 3.3456x; 1.1762x over previous
import functools

import jax
import jax.numpy as jnp
from jax import lax
from jax.experimental import pallas as pl
from jax.experimental.pallas import tpu as pltpu
from jax.experimental.pallas import tpu_sc as plsc

N = 10000
E = 320000
D = 128
L = 16
NC = 2
NS = 16
NW = NC * NS
EPW = E // NW
CH = 40
NCH = EPW // CH
CW = 16
NPAD = 10240
NPT = NPAD // NS

_MESH = dict(core_axis_name="c", subcore_axis_name="s")


@functools.partial(
    pl.kernel,
    mesh=plsc.VectorSubcoreMesh(**_MESH),
    out_type=(jax.ShapeDtypeStruct((E, D), jnp.float32),
              jax.ShapeDtypeStruct((NC, NPAD, D), jnp.float32)),
    scratch_types=[
        pltpu.VMEM((CH,), jnp.int32),
        pltpu.VMEM((CH,), jnp.int32),
        pltpu.VMEM((CH,), jnp.int32),
        pltpu.VMEM((CH,), jnp.int32),
        pltpu.VMEM((CH, D), jnp.float32),
        pltpu.VMEM((CH, D), jnp.float32),
        pltpu.VMEM((CH, D), jnp.float32),
        pltpu.VMEM((CH, D), jnp.float32),
        pltpu.VMEM((CH, D), jnp.float32),
        pltpu.VMEM_SHARED((NPAD, D), jnp.float32),
        pltpu.SemaphoreType.DMA,
        pltpu.SemaphoreType.DMA,
        pltpu.SemaphoreType.DMA,
        pltpu.SemaphoreType.DMA,
        pltpu.SemaphoreType.DMA,
        pltpu.SemaphoreType.DMA,
        pltpu.SemaphoreType.DMA,
        pltpu.SemaphoreType.DMA,
    ],
)
def _sc_gather(xs_hbm, xr_hbm, send_hbm, recv_hbm, g_hbm, cnt_hbm,
               sidx0, sidx1, ridx0, ridx1, bufa0, bufa1, bufb0, bufb1, ones,
               scnt,
               sema0, sema1, semb0, semb1, semo0, semo1, semw0, semw1):
    cid = lax.axis_index("c")
    sid = lax.axis_index("s")
    wid = sid * NC + cid
    base = wid * EPW
    zero16 = jnp.zeros((L,), jnp.float32)
    one16 = jnp.ones((L,), jnp.float32)
    sets = ((sidx0, ridx0, bufa0, bufb0, sema0, semb0, semo0, semw0),
            (sidx1, ridx1, bufa1, bufb1, sema1, semb1, semo1, semw1))

    def fill_body(r, c):
        for j in range(D // L):
            bufa0[r, pl.ds(j * L, L)] = zero16
            ones[r, pl.ds(j * L, L)] = one16
        return c

    lax.fori_loop(0, CH, fill_body, 0)

    nbase = sid * NPT
    for k in range(NPT // CH):
        pltpu.sync_copy(bufa0, scnt.at[pl.ds(nbase + k * CH, CH)])

    def load_and_fire(c, s):
        sidx, ridx, bufa, bufb, sema, semb, _, _ = s
        off = pl.multiple_of(base + c * CH, 8)
        pltpu.sync_copy(send_hbm.at[pl.ds(off, CH)], sidx)
        pltpu.sync_copy(recv_hbm.at[pl.ds(off, CH)], ridx)
        pltpu.async_copy(xs_hbm.at[sidx], bufa, sema)
        pltpu.async_copy(xr_hbm.at[ridx], bufb, semb)

    for k in range(2):
        load_and_fire(k, sets[k])
    plsc.subcore_barrier()

    def pair_body(si, carry):
        for k in range(2):
            sidx, ridx, bufa, bufb, sema, semb, semo, semw = sets[k]
            c = 2 * si + k
            off = pl.multiple_of(base + c * CH, 8)
            pltpu.make_async_copy(xs_hbm.at[sidx], bufa, sema).wait()
            pltpu.make_async_copy(xr_hbm.at[ridx], bufb, semb).wait()
            cpo = pltpu.async_copy(ones, scnt.at[ridx], semo, add=True)

            def row_body(r, c2):
                for j in range(D // L):
                    sl = pl.ds(j * L, L)
                    bufa[r, sl] = bufa[r, sl] + bufb[r, sl]
                return c2

            lax.fori_loop(0, CH, row_body, 0)
            cpw = pltpu.async_copy(bufa, g_hbm.at[pl.ds(off, CH)], semw)
            cpo.wait()
            cpw.wait()
            load_and_fire(jnp.minimum(c + 2, NCH - 1), sets[k])
        return carry

    lax.fori_loop(0, NCH // 2, pair_body, 0)
    for k in range(2):
        sidx, ridx, bufa, bufb, sema, semb, _, _ = sets[k]
        pltpu.make_async_copy(xs_hbm.at[sidx], bufa, sema).wait()
        pltpu.make_async_copy(xr_hbm.at[ridx], bufb, semb).wait()
    plsc.subcore_barrier()

    for k in range(NPT // CH):
        sl = pl.ds(nbase + k * CH, CH)
        pltpu.sync_copy(scnt.at[sl], bufa0)
        pltpu.sync_copy(bufa0, cnt_hbm.at[cid, sl])


@functools.partial(
    pl.kernel,
    mesh=plsc.VectorSubcoreMesh(**_MESH),
    out_type=jax.ShapeDtypeStruct((NC, NPAD, D), jnp.float32),
    scratch_types=[
        pltpu.VMEM((CH,), jnp.int32),
        pltpu.VMEM((CH,), jnp.int32),
        pltpu.VMEM((CH, D), jnp.float32),
        pltpu.VMEM((CH, D), jnp.float32),
        pltpu.VMEM_SHARED((NPAD, D), jnp.float32),
        pltpu.SemaphoreType.DMA,
        pltpu.SemaphoreType.DMA,
        pltpu.SemaphoreType.DMA,
        pltpu.SemaphoreType.DMA,
    ],
)
def _sc_scatter(h_hbm, recv_hbm, sum_hbm,
                ridx0, ridx1, hbuf0, hbuf1, ssum, semh0, semh1, sems0, sems1):
    cid = lax.axis_index("c")
    sid = lax.axis_index("s")
    zero16 = jnp.zeros((L,), jnp.float32)
    sets = ((ridx0, hbuf0, semh0, sems0), (ridx1, hbuf1, semh1, sems1))

    def zrow_body(r, c):
        for j in range(D // L):
            hbuf0[r, pl.ds(j * L, L)] = zero16
        return c

    lax.fori_loop(0, CH, zrow_body, 0)

    nbase = sid * NPT
    for k in range(NPT // CH):
        pltpu.sync_copy(hbuf0, ssum.at[pl.ds(nbase + k * CH, CH)])

    base = (sid * NC + cid) * EPW

    def load_and_fire(c, s):
        ridx, hbuf, semh, _ = s
        off = pl.multiple_of(base + c * CH, 8)
        pltpu.sync_copy(recv_hbm.at[pl.ds(off, CH)], ridx)
        pltpu.async_copy(h_hbm.at[pl.ds(off, CH)], hbuf, semh)

    for k in range(2):
        load_and_fire(k, sets[k])
    plsc.subcore_barrier()

    def pair_body(si, carry):
        for k in range(2):
            ridx, hbuf, semh, sems = sets[k]
            c = 2 * si + k
            pltpu.make_async_copy(h_hbm.at[pl.ds(0, CH)], hbuf, semh).wait()
            cps = pltpu.async_copy(hbuf, ssum.at[ridx], sems, add=True)
            cps.wait()
            load_and_fire(jnp.minimum(c + 2, NCH - 1), sets[k])
        return carry

    lax.fori_loop(0, NCH // 2, pair_body, 0)
    for k in range(2):
        ridx, hbuf, semh, _ = sets[k]
        pltpu.make_async_copy(h_hbm.at[pl.ds(0, CH)], hbuf, semh).wait()
    plsc.subcore_barrier()

    for k in range(NPT // CH):
        sl = pl.ds(nbase + k * CH, CH)
        pltpu.sync_copy(ssum.at[sl], hbuf0)
        pltpu.sync_copy(hbuf0, sum_hbm.at[cid, sl])


def _tc_node_transform(x2, wa, wb):
    TN = 2000

    def body(x_ref, wa_ref, wb_ref, xs_ref, xr_ref):
        xv = x_ref[...]
        xs_ref[...] = jnp.dot(xv, wa_ref[...], preferred_element_type=jnp.float32)
        xr_ref[...] = jnp.dot(xv, wb_ref[...], preferred_element_type=jnp.float32)

    return pl.pallas_call(
        body,
        grid=(N // TN,),
        in_specs=[
            pl.BlockSpec((TN, D), lambda i: (i, 0)),
            pl.BlockSpec((D, D), lambda i: (0, 0)),
            pl.BlockSpec((D, D), lambda i: (0, 0)),
        ],
        out_specs=[
            pl.BlockSpec((TN, D), lambda i: (i, 0)),
            pl.BlockSpec((TN, D), lambda i: (i, 0)),
        ],
        out_shape=[jax.ShapeDtypeStruct((N, D), jnp.float32)] * 2,
    )(x2, wa, wb)


def _tc_message(g, ea2, wc, b1, w2, b2):
    TE = 2000

    def body(g_ref, ea_ref, wc_ref, b1_ref, w2_ref, b2_ref, h_ref):
        pre = (g_ref[...]
               + jnp.dot(ea_ref[...], wc_ref[...], preferred_element_type=jnp.float32)
               + b1_ref[...])
        h1 = pre * jax.nn.sigmoid(pre)
        pre2 = jnp.dot(h1, w2_ref[...], preferred_element_type=jnp.float32) + b2_ref[...]
        h_ref[...] = pre2 * jax.nn.sigmoid(pre2)

    return pl.pallas_call(
        body,
        grid=(E // TE,),
        in_specs=[
            pl.BlockSpec((TE, D), lambda i: (i, 0)),
            pl.BlockSpec((TE, D), lambda i: (i, 0)),
            pl.BlockSpec((D, D), lambda i: (0, 0)),
            pl.BlockSpec((1, D), lambda i: (0, 0)),
            pl.BlockSpec((D, D), lambda i: (0, 0)),
            pl.BlockSpec((1, D), lambda i: (0, 0)),
        ],
        out_specs=pl.BlockSpec((TE, D), lambda i: (i, 0)),
        out_shape=jax.ShapeDtypeStruct((E, D), jnp.float32),
    )(g, ea2, wc, b1, w2, b2)


def _tc_update(x2, psum, pcnt, wu1, bu1, wu2, bu2):
    TN = 2000

    def body(x_ref, ps_ref, pc_ref, wu1_ref, bu1_ref, wu2_ref, bu2_ref, o_ref):
        s = ps_ref[0] + ps_ref[1]
        c = pc_ref[0, :, 0:1] + pc_ref[1, :, 0:1]
        agg = s / jnp.maximum(c, 1.0)
        xn = x_ref[...] + agg
        t = jnp.dot(xn, wu1_ref[...], preferred_element_type=jnp.float32) + bu1_ref[...]
        t = t * jax.nn.sigmoid(t)
        u = jnp.dot(t, wu2_ref[...], preferred_element_type=jnp.float32) + bu2_ref[...]
        o_ref[...] = xn + u

    return pl.pallas_call(
        body,
        grid=(N // TN,),
        in_specs=[
            pl.BlockSpec((TN, D), lambda i: (i, 0)),
            pl.BlockSpec((NC, TN, D), lambda i: (0, i, 0)),
            pl.BlockSpec((NC, TN, D), lambda i: (0, i, 0)),
            pl.BlockSpec((D, 2 * D), lambda i: (0, 0)),
            pl.BlockSpec((1, 2 * D), lambda i: (0, 0)),
            pl.BlockSpec((2 * D, D), lambda i: (0, 0)),
            pl.BlockSpec((1, D), lambda i: (0, 0)),
        ],
        out_specs=pl.BlockSpec((TN, D), lambda i: (i, 0)),
        out_shape=jax.ShapeDtypeStruct((N, D), jnp.float32),
    )(x2, psum, pcnt, wu1, bu1, wu2, bu2)


def kernel(x, edge_attr, edges, W_m1, b_m1, W_m2, b_m2, W_u1, b_u1, W_u2, b_u2):
    x2 = x[0]
    ea2 = edge_attr[0]
    send = edges[0]
    recv = edges[1]
    wa = W_m1[:D]
    wb = W_m1[D:2 * D]
    wc = W_m1[2 * D:]
    b1 = b_m1.reshape(1, D)
    b2 = b_m2.reshape(1, D)
    bu1 = b_u1.reshape(1, 2 * D)
    bu2 = b_u2.reshape(1, D)

    xs, xr = _tc_node_transform(x2, wa, wb)
    g, pcnt = _sc_gather(xs, xr, send, recv)
    h = _tc_message(g, ea2, wc, b1, W_m2, b2)
    psum = _sc_scatter(h, recv)
    xo = _tc_update(x2, psum, pcnt, W_u1, bu1, W_u2, bu2)
    return (xo[None], h[None])
